# baseline (speedup 1.0000x reference)
import functools

import jax
import jax.numpy as jnp
from jax import lax
from jax.experimental import pallas as pl
from jax.experimental.pallas import tpu as pltpu
from jax.experimental.pallas import tpu_sc as plsc

N_VOCAB = 100000
EMBED_DIM = 64
BATCH = 4096
HIST = 50

NC = 2
NS = 16
NW = NC * NS

ROWS_W = BATCH // NW
K = 8
G = ROWS_W // K

_mesh = plsc.VectorSubcoreMesh(core_axis_name="c", subcore_axis_name="s")


@functools.partial(
    pl.kernel,
    mesh=_mesh,
    out_type=jax.ShapeDtypeStruct((BATCH, HIST, EMBED_DIM), jnp.float32),
    scratch_types=[
        pltpu.VMEM((ROWS_W, HIST), jnp.int32),
        pltpu.VMEM((2, K, HIST, EMBED_DIM), jnp.float32),
        [pltpu.SemaphoreType.DMA] * 2,
        [pltpu.SemaphoreType.DMA] * 2,
    ],
    compiler_params=pltpu.CompilerParams(use_tc_tiling_on_sc=False),
)
def _embed_lookup(x_hbm, table_hbm, out_hbm, idx_v, rows_v, gsems, ssems):
    wid = lax.axis_index("s") * NC + lax.axis_index("c")
    base = wid * ROWS_W
    pltpu.sync_copy(x_hbm.at[pl.ds(base, ROWS_W)], idx_v)

    def fire(g, b):
        for k in range(K):
            pltpu.async_copy(
                table_hbm.at[idx_v.at[g * K + k]], rows_v.at[b, k], gsems[b])

    def wait_gathers(g, b):
        for k in range(K):
            pltpu.make_async_copy(
                table_hbm.at[idx_v.at[g * K + k]], rows_v.at[b, k],
                gsems[b]).wait()

    def start_store(g, b):
        pltpu.async_copy(
            rows_v.at[b], out_hbm.at[pl.ds(base + g * K, K)], ssems[b])

    def wait_store(g, b):
        pltpu.make_async_copy(
            rows_v.at[b], out_hbm.at[pl.ds(base + g * K, K)], ssems[b]).wait()

    fire(0, 0)
    fire(1, 1)

    def body(p, carry):
        for b in (0, 1):
            g = 2 * p + b
            wait_gathers(g, b)
            start_store(g, b)
            wait_store(g, b)
            fire(g + 2, b)
        return carry

    lax.fori_loop(0, G // 2 - 1, body, 0)

    for b in (0, 1):
        g = G - 2 + b
        wait_gathers(g, b)
        start_store(g, b)
        wait_store(g, b)


def kernel(x, weight):
    return _embed_lookup(x.astype(jnp.int32), weight)
